# Initial kernel scaffold; baseline (speedup 1.0000x reference)
import jax
import jax.numpy as jnp
from jax.experimental import pallas as pl


def kernel(x, y, embed, conv_w3, conv_b3, conv_w4, conv_b4, conv_w5, conv_b5, fc_w, fc_b, mem_K, mem_V):
    raise NotImplementedError("write your pallas kernel here")


# speedup vs baseline: 66.7115x; 66.7115x over previous
import functools

import jax
import jax.numpy as jnp
from jax import lax
from jax.experimental import pallas as pl
from jax.experimental.pallas import tpu as pltpu
from jax.experimental.pallas import tpu_sc as plsc

B = 1024
L = 50
D = 128
KEY = 128
KN = 100
KNP = 128
MEM = 100000
CBLK = 1024
NBLK = 98
MEMP = CBLK * NBLK
BT = 64
ALPHA = 0.1

NW = 32
GCH = 320


def _sc_gather(embed, idx_flat):
    n = idx_flat.shape[0]
    bpw = n // NW
    mesh = plsc.VectorSubcoreMesh(core_axis_name="c", subcore_axis_name="s")

    @functools.partial(
        pl.kernel,
        mesh=mesh,
        out_type=jax.ShapeDtypeStruct((n, D), jnp.float32),
        scratch_types=[
            pltpu.VMEM((GCH,), jnp.int32),
            pltpu.VMEM((GCH, D), jnp.float32),
            pltpu.SemaphoreType.DMA,
        ],
    )
    def gather_kernel(table_hbm, idx_hbm, out_hbm, idx_v, rows_v, sem):
        wid = lax.axis_index("s") * 2 + lax.axis_index("c")
        base = wid * bpw

        @pl.loop(0, bpw, step=GCH)
        def _(off):
            pltpu.sync_copy(idx_hbm.at[pl.ds(base + off, GCH)], idx_v)
            pltpu.async_copy(table_hbm.at[idx_v], rows_v, sem).wait()
            pltpu.sync_copy(rows_v, out_hbm.at[pl.ds(base + off, GCH)])

    return gather_kernel(embed, idx_flat)


def _q_body(e_ref, w3_ref, b3_ref, w4_ref, b4_ref, w5_ref, b5_ref,
            fcw_ref, fcb_ref, q_ref):
    e2 = e_ref[...].reshape(BT * L, D)
    feats = []
    for w_ref, b_ref, s in ((w3_ref, b3_ref, 3), (w4_ref, b4_ref, 4),
                            (w5_ref, b5_ref, 5)):
        t = L - s + 1
        acc = None
        for dt in range(s):
            p = jnp.dot(e2, w_ref[dt], preferred_element_type=jnp.float32)
            sl = p.reshape(BT, L, KNP)[:, dt:dt + t, :]
            acc = sl if acc is None else acc + sl
        c = jnp.maximum(acc + b_ref[...][None, :, :], 0.0)
        feats.append(jnp.max(c, axis=1))
    q = jnp.zeros((BT, KEY), jnp.float32) + fcb_ref[...]
    for i, f in enumerate(feats):
        q = q + jnp.dot(f, fcw_ref[pl.ds(i * KNP, KNP), :],
                        preferred_element_type=jnp.float32)
    norm = jnp.sqrt(jnp.sum(q * q, axis=1, keepdims=True))
    q_ref[...] = q / (norm + 1e-8)


def _q_from_e(e3, w3p, b3p, w4p, b4p, w5p, b5p, fcwp, fcbp):
    grid = (B // BT,)
    return pl.pallas_call(
        _q_body,
        grid=grid,
        in_specs=[
            pl.BlockSpec((BT, L, D), lambda i: (i, 0, 0)),
            pl.BlockSpec((3, D, KNP), lambda i: (0, 0, 0)),
            pl.BlockSpec((1, KNP), lambda i: (0, 0)),
            pl.BlockSpec((4, D, KNP), lambda i: (0, 0, 0)),
            pl.BlockSpec((1, KNP), lambda i: (0, 0)),
            pl.BlockSpec((5, D, KNP), lambda i: (0, 0, 0)),
            pl.BlockSpec((1, KNP), lambda i: (0, 0)),
            pl.BlockSpec((3 * KNP, KEY), lambda i: (0, 0)),
            pl.BlockSpec((1, KEY), lambda i: (0, 0)),
        ],
        out_specs=pl.BlockSpec((BT, KEY), lambda i: (i, 0)),
        out_shape=jax.ShapeDtypeStruct((B, KEY), jnp.float32),
    )(e3, w3p, b3p, w4p, b4p, w5p, b5p, fcwp, fcbp)


def _mem_body(y_ref, q_ref, kt_ref, v_ref, loss_ref, acc_ref, oth_ref,
              pos_ref, neg_ref, cnt_ref):
    p = pl.program_id(0)
    j = pl.program_id(1)
    sims = jnp.dot(q_ref[...], kt_ref[...],
                   preferred_element_type=jnp.float32)
    memv = v_ref[0]
    y = y_ref[...]
    match = y == memv
    col = j * CBLK + lax.broadcasted_iota(jnp.int32, (1, CBLK), 1)
    valid = col < MEM

    @pl.when(jnp.logical_and(p == 0, j == 0))
    def _():
        pos_ref[...] = jnp.full((B, 1), -1e9, jnp.float32)
        neg_ref[...] = jnp.full((B, 1), -1e9, jnp.float32)
        cnt_ref[...] = jnp.zeros((B, 1), jnp.float32)

    @pl.when(p == 0)
    def _():
        posb = jnp.max(jnp.where(match & valid, sims, -1e9), axis=1,
                       keepdims=True)
        negb = jnp.max(jnp.where(jnp.logical_and(~match, valid), sims, -1e9),
                       axis=1, keepdims=True)
        pos_ref[...] = jnp.maximum(pos_ref[...], posb)
        neg_ref[...] = jnp.maximum(neg_ref[...], negb)

    @pl.when(p == 1)
    def _():
        gt = jnp.logical_and(sims > pos_ref[...],
                             jnp.logical_and(~match, valid))
        cnt_ref[...] += jnp.sum(gt.astype(jnp.float32), axis=1, keepdims=True)

    @pl.when(jnp.logical_and(p == 1, j == NBLK - 1))
    def _():
        pos = pos_ref[...]
        neg = neg_ref[...]
        cnt = cnt_ref[...]
        loss_ref[...] = jnp.mean(
            jnp.maximum(neg - pos + ALPHA, 0.0)).reshape(1, 1)
        acc_ref[...] = jnp.mean((cnt < 1.0).astype(jnp.float32)).reshape(1, 1)
        ii = lax.broadcasted_iota(jnp.int32, (1, 3), 1).astype(jnp.float32)
        thr = (ii + 1.0) * (ii + 1.0) + 1.0
        oth_ref[...] = jnp.mean((cnt < thr).astype(jnp.float32), axis=0,
                                keepdims=True)


def _mem_stats(y2, q, kt, v3):
    grid = (2, NBLK)
    return pl.pallas_call(
        _mem_body,
        grid=grid,
        in_specs=[
            pl.BlockSpec((B, 1), lambda p, j: (0, 0)),
            pl.BlockSpec((B, KEY), lambda p, j: (0, 0)),
            pl.BlockSpec((KEY, CBLK), lambda p, j: (0, j)),
            pl.BlockSpec((1, 1, CBLK), lambda p, j: (j, 0, 0)),
        ],
        out_specs=[
            pl.BlockSpec((1, 1), lambda p, j: (0, 0)),
            pl.BlockSpec((1, 1), lambda p, j: (0, 0)),
            pl.BlockSpec((1, 3), lambda p, j: (0, 0)),
        ],
        out_shape=[
            jax.ShapeDtypeStruct((1, 1), jnp.float32),
            jax.ShapeDtypeStruct((1, 1), jnp.float32),
            jax.ShapeDtypeStruct((1, 3), jnp.float32),
        ],
        scratch_shapes=[
            pltpu.VMEM((B, 1), jnp.float32),
            pltpu.VMEM((B, 1), jnp.float32),
            pltpu.VMEM((B, 1), jnp.float32),
        ],
        compiler_params=pltpu.CompilerParams(
            dimension_semantics=("arbitrary", "arbitrary")),
    )(y2, q, kt, v3)


def kernel(x, y, embed, conv_w3, conv_b3, conv_w4, conv_b4, conv_w5, conv_b5,
           fc_w, fc_b, mem_K, mem_V):
    xf = x.reshape(B * L).astype(jnp.int32)
    pad_ch = KNP - KN
    w3p = jnp.pad(conv_w3, ((0, 0), (0, 0), (0, pad_ch)))
    w4p = jnp.pad(conv_w4, ((0, 0), (0, 0), (0, pad_ch)))
    w5p = jnp.pad(conv_w5, ((0, 0), (0, 0), (0, pad_ch)))
    b3p = jnp.pad(conv_b3, (0, pad_ch)).reshape(1, KNP)
    b4p = jnp.pad(conv_b4, (0, pad_ch)).reshape(1, KNP)
    b5p = jnp.pad(conv_b5, (0, pad_ch)).reshape(1, KNP)
    fcw = fc_w.reshape(3, KN, KEY)
    fcwp = jnp.pad(fcw, ((0, 0), (0, pad_ch), (0, 0))).reshape(3 * KNP, KEY)
    fcbp = fc_b.reshape(1, KEY)
    kt = jnp.pad(mem_K, ((0, MEMP - MEM), (0, 0))).T
    v3 = jnp.pad(mem_V.astype(jnp.int32), (0, MEMP - MEM),
                 constant_values=-1).reshape(NBLK, 1, CBLK)
    y2 = y.astype(jnp.int32).reshape(B, 1)

    e = _sc_gather(embed, xf)
    e3 = e.reshape(B, L, D)

    q = _q_from_e(e3, w3p, b3p, w4p, b4p, w5p, b5p, fcwp, fcbp)

    loss2, acc2, oth2 = _mem_stats(y2, q, kt, v3)
    return (loss2.reshape(()), acc2.reshape(()), oth2.reshape(3))


# speedup vs baseline: 95.2481x; 1.4278x over previous
import functools

import jax
import jax.numpy as jnp
from jax import lax
from jax.experimental import pallas as pl
from jax.experimental.pallas import tpu as pltpu
from jax.experimental.pallas import tpu_sc as plsc

B = 1024
L = 50
D = 128
KEY = 128
KN = 100
KNP = 128
MEM = 100000
CBLK = 2048
NBLK = 49
MEMP = CBLK * NBLK
BT = 64
ALPHA = 0.1

NW = 32
GCH = 320


def _sc_gather(embed, idx_flat):
    n = idx_flat.shape[0]
    bpw = n // NW
    mesh = plsc.VectorSubcoreMesh(core_axis_name="c", subcore_axis_name="s")

    @functools.partial(
        pl.kernel,
        mesh=mesh,
        out_type=jax.ShapeDtypeStruct((n, D), jnp.float32),
        scratch_types=[
            pltpu.VMEM((GCH,), jnp.int32),
            pltpu.VMEM((GCH, D), jnp.float32),
            pltpu.SemaphoreType.DMA,
        ],
    )
    def gather_kernel(table_hbm, idx_hbm, out_hbm, idx_v, rows_v, sem):
        wid = lax.axis_index("s") * 2 + lax.axis_index("c")
        base = wid * bpw

        @pl.loop(0, bpw, step=GCH)
        def _(off):
            pltpu.sync_copy(idx_hbm.at[pl.ds(base + off, GCH)], idx_v)
            pltpu.async_copy(table_hbm.at[idx_v], rows_v, sem).wait()
            pltpu.sync_copy(rows_v, out_hbm.at[pl.ds(base + off, GCH)])

    return gather_kernel(embed, idx_flat)


def _q_body(e_ref, w3_ref, b3_ref, w4_ref, b4_ref, w5_ref, b5_ref,
            fcw_ref, fcb_ref, q_ref):
    e2 = e_ref[...].reshape(BT * L, D)
    feats = []
    for w_ref, b_ref, s in ((w3_ref, b3_ref, 3), (w4_ref, b4_ref, 4),
                            (w5_ref, b5_ref, 5)):
        t = L - s + 1
        acc = None
        for dt in range(s):
            p = jnp.dot(e2, w_ref[dt], preferred_element_type=jnp.float32)
            sl = p.reshape(BT, L, KNP)[:, dt:dt + t, :]
            acc = sl if acc is None else acc + sl
        c = jnp.maximum(acc + b_ref[...][None, :, :], 0.0)
        feats.append(jnp.max(c, axis=1))
    q = jnp.zeros((BT, KEY), jnp.float32) + fcb_ref[...]
    for i, f in enumerate(feats):
        q = q + jnp.dot(f, fcw_ref[pl.ds(i * KNP, KNP), :],
                        preferred_element_type=jnp.float32)
    norm = jnp.sqrt(jnp.sum(q * q, axis=1, keepdims=True))
    q_ref[...] = q / (norm + 1e-8)


def _q_from_e(e3, w3p, b3p, w4p, b4p, w5p, b5p, fcwp, fcbp):
    grid = (B // BT,)
    return pl.pallas_call(
        _q_body,
        grid=grid,
        in_specs=[
            pl.BlockSpec((BT, L, D), lambda i: (i, 0, 0)),
            pl.BlockSpec((3, D, KNP), lambda i: (0, 0, 0)),
            pl.BlockSpec((1, KNP), lambda i: (0, 0)),
            pl.BlockSpec((4, D, KNP), lambda i: (0, 0, 0)),
            pl.BlockSpec((1, KNP), lambda i: (0, 0)),
            pl.BlockSpec((5, D, KNP), lambda i: (0, 0, 0)),
            pl.BlockSpec((1, KNP), lambda i: (0, 0)),
            pl.BlockSpec((3 * KNP, KEY), lambda i: (0, 0)),
            pl.BlockSpec((1, KEY), lambda i: (0, 0)),
        ],
        out_specs=pl.BlockSpec((BT, KEY), lambda i: (i, 0)),
        out_shape=jax.ShapeDtypeStruct((B, KEY), jnp.float32),
    )(e3, w3p, b3p, w4p, b4p, w5p, b5p, fcwp, fcbp)


def _mem_body(y_ref, q_ref, kt_ref, v_ref, loss_ref, acc_ref, oth_ref,
              pos_ref, neg_ref, cnt_ref):
    p = pl.program_id(0)
    j = pl.program_id(1)
    raw = lax.dot_general(q_ref[...], kt_ref[...],
                          (((1,), (1,)), ((), ())),
                          preferred_element_type=jnp.float32)
    if MEMP - MEM < CBLK:
        col = lax.broadcasted_iota(jnp.int32, (1, CBLK), 1)
        sims = jnp.where(
            jnp.logical_or(j < NBLK - 1, col < MEM - (NBLK - 1) * CBLK),
            raw, -1e9)
    else:
        sims = raw

    @pl.when(jnp.logical_and(p == 0, j == 0))
    def _():
        pos_ref[...] = jnp.full((B, 1), -1e9, jnp.float32)
        neg_ref[...] = jnp.full((B, 1), -1e9, jnp.float32)
        cnt_ref[...] = jnp.zeros((B, 1), jnp.float32)

    @pl.when(p == 0)
    def _():
        memv = v_ref[0]
        match = y_ref[...] == memv
        posb = jnp.max(jnp.where(match, sims, -1e9), axis=1, keepdims=True)
        negb = jnp.max(jnp.where(match, -1e9, sims), axis=1, keepdims=True)
        pos_ref[...] = jnp.maximum(pos_ref[...], posb)
        neg_ref[...] = jnp.maximum(neg_ref[...], negb)

    @pl.when(p == 1)
    def _():
        gt = sims > pos_ref[...]
        cnt_ref[...] += jnp.sum(gt.astype(jnp.float32), axis=1, keepdims=True)

    @pl.when(jnp.logical_and(p == 1, j == NBLK - 1))
    def _():
        pos = pos_ref[...]
        neg = neg_ref[...]
        cnt = cnt_ref[...]
        loss_ref[...] = jnp.mean(
            jnp.maximum(neg - pos + ALPHA, 0.0)).reshape(1, 1)
        acc_ref[...] = jnp.mean((cnt < 1.0).astype(jnp.float32)).reshape(1, 1)
        ii = lax.broadcasted_iota(jnp.int32, (1, 3), 1).astype(jnp.float32)
        thr = (ii + 1.0) * (ii + 1.0) + 1.0
        oth_ref[...] = jnp.mean((cnt < thr).astype(jnp.float32), axis=0,
                                keepdims=True)


def _mem_stats(y2, q, kt, v3):
    grid = (2, NBLK)
    return pl.pallas_call(
        _mem_body,
        grid=grid,
        in_specs=[
            pl.BlockSpec((B, 1), lambda p, j: (0, 0)),
            pl.BlockSpec((B, KEY), lambda p, j: (0, 0)),
            pl.BlockSpec((CBLK, KEY), lambda p, j: (j, 0)),
            pl.BlockSpec((1, 1, CBLK), lambda p, j: (j, 0, 0)),
        ],
        out_specs=[
            pl.BlockSpec((1, 1), lambda p, j: (0, 0)),
            pl.BlockSpec((1, 1), lambda p, j: (0, 0)),
            pl.BlockSpec((1, 3), lambda p, j: (0, 0)),
        ],
        out_shape=[
            jax.ShapeDtypeStruct((1, 1), jnp.float32),
            jax.ShapeDtypeStruct((1, 1), jnp.float32),
            jax.ShapeDtypeStruct((1, 3), jnp.float32),
        ],
        scratch_shapes=[
            pltpu.VMEM((B, 1), jnp.float32),
            pltpu.VMEM((B, 1), jnp.float32),
            pltpu.VMEM((B, 1), jnp.float32),
        ],
        compiler_params=pltpu.CompilerParams(
            dimension_semantics=("arbitrary", "arbitrary")),
    )(y2, q, kt, v3)


def kernel(x, y, embed, conv_w3, conv_b3, conv_w4, conv_b4, conv_w5, conv_b5,
           fc_w, fc_b, mem_K, mem_V):
    xf = x.reshape(B * L).astype(jnp.int32)
    pad_ch = KNP - KN
    w3p = jnp.pad(conv_w3, ((0, 0), (0, 0), (0, pad_ch)))
    w4p = jnp.pad(conv_w4, ((0, 0), (0, 0), (0, pad_ch)))
    w5p = jnp.pad(conv_w5, ((0, 0), (0, 0), (0, pad_ch)))
    b3p = jnp.pad(conv_b3, (0, pad_ch)).reshape(1, KNP)
    b4p = jnp.pad(conv_b4, (0, pad_ch)).reshape(1, KNP)
    b5p = jnp.pad(conv_b5, (0, pad_ch)).reshape(1, KNP)
    fcw = fc_w.reshape(3, KN, KEY)
    fcwp = jnp.pad(fcw, ((0, 0), (0, pad_ch), (0, 0))).reshape(3 * KNP, KEY)
    fcbp = fc_b.reshape(1, KEY)
    kt = jnp.pad(mem_K, ((0, MEMP - MEM), (0, 0)))
    v3 = jnp.pad(mem_V.astype(jnp.int32), (0, MEMP - MEM),
                 constant_values=-1).reshape(NBLK, 1, CBLK)
    y2 = y.astype(jnp.int32).reshape(B, 1)

    e = _sc_gather(embed, xf)
    e3 = e.reshape(B, L, D)

    q = _q_from_e(e3, w3p, b3p, w4p, b4p, w5p, b5p, fcwp, fcbp)

    loss2, acc2, oth2 = _mem_stats(y2, q, kt, v3)
    return (loss2.reshape(()), acc2.reshape(()), oth2.reshape(3))


# speedup vs baseline: 102.1366x; 1.0723x over previous
import functools

import jax
import jax.numpy as jnp
from jax import lax
from jax.experimental import pallas as pl
from jax.experimental.pallas import tpu as pltpu
from jax.experimental.pallas import tpu_sc as plsc

B = 1024
L = 50
D = 128
KEY = 128
KN = 100
KNP = 128
MEM = 100000
CBLK = 2048
NBLK = 49
MEMP = CBLK * NBLK
BT = 64
ALPHA = 0.1

NW = 32
GCH = 320


def _sc_gather(embed, idx_flat):
    n = idx_flat.shape[0]
    bpw = n // NW
    mesh = plsc.VectorSubcoreMesh(core_axis_name="c", subcore_axis_name="s")

    @functools.partial(
        pl.kernel,
        mesh=mesh,
        out_type=jax.ShapeDtypeStruct((n, D), jnp.float32),
        scratch_types=[
            pltpu.VMEM((GCH,), jnp.int32),
            pltpu.VMEM((GCH, D), jnp.float32),
            pltpu.SemaphoreType.DMA,
        ],
    )
    def gather_kernel(table_hbm, idx_hbm, out_hbm, idx_v, rows_v, sem):
        wid = lax.axis_index("s") * 2 + lax.axis_index("c")
        base = wid * bpw

        @pl.loop(0, bpw, step=GCH)
        def _(off):
            pltpu.sync_copy(idx_hbm.at[pl.ds(base + off, GCH)], idx_v)
            pltpu.async_copy(table_hbm.at[idx_v], rows_v, sem).wait()
            pltpu.sync_copy(rows_v, out_hbm.at[pl.ds(base + off, GCH)])

    return gather_kernel(embed, idx_flat)


def _q_body(e_ref, wcat_ref, b3_ref, b4_ref, b5_ref, fcw_ref, fcb_ref,
            q_ref):
    e2 = e_ref[...].reshape(BT * L, D)
    g = jnp.dot(e2, wcat_ref[...],
                preferred_element_type=jnp.float32)
    g3 = g.reshape(BT, L, 12 * KNP)
    feats = []
    off = 0
    for b_ref, s in ((b3_ref, 3), (b4_ref, 4), (b5_ref, 5)):
        t = L - s + 1
        acc = None
        for dt in range(s):
            sl = g3[:, dt:dt + t, off:off + KNP]
            acc = sl if acc is None else acc + sl
            off += KNP
        c = jnp.maximum(acc + b_ref[...][None, :, :], 0.0)
        feats.append(jnp.max(c, axis=1))
    q = jnp.zeros((BT, KEY), jnp.float32) + fcb_ref[...]
    for i, f in enumerate(feats):
        q = q + jnp.dot(f, fcw_ref[pl.ds(i * KNP, KNP), :],
                        preferred_element_type=jnp.float32)
    norm = jnp.sqrt(jnp.sum(q * q, axis=1, keepdims=True))
    q_ref[...] = q / (norm + 1e-8)


def _q_from_e(e3, wcat, b3p, b4p, b5p, fcwp, fcbp):
    grid = (B // BT,)
    return pl.pallas_call(
        _q_body,
        grid=grid,
        in_specs=[
            pl.BlockSpec((BT, L, D), lambda i: (i, 0, 0)),
            pl.BlockSpec((D, 12 * KNP), lambda i: (0, 0)),
            pl.BlockSpec((1, KNP), lambda i: (0, 0)),
            pl.BlockSpec((1, KNP), lambda i: (0, 0)),
            pl.BlockSpec((1, KNP), lambda i: (0, 0)),
            pl.BlockSpec((3 * KNP, KEY), lambda i: (0, 0)),
            pl.BlockSpec((1, KEY), lambda i: (0, 0)),
        ],
        out_specs=pl.BlockSpec((BT, KEY), lambda i: (i, 0)),
        out_shape=jax.ShapeDtypeStruct((B, KEY), jnp.float32),
    )(e3, wcat, b3p, b4p, b5p, fcwp, fcbp)


def _mem_body(y_ref, q_ref, kt_ref, v_ref, loss_ref, acc_ref, oth_ref,
              pos_ref, neg_ref, cnt_ref):
    p = pl.program_id(0)
    j = pl.program_id(1)
    raw = lax.dot_general(q_ref[...], kt_ref[...],
                          (((1,), (1,)), ((), ())),
                          preferred_element_type=jnp.float32)
    if MEMP - MEM < CBLK:
        col = lax.broadcasted_iota(jnp.int32, (1, CBLK), 1)
        sims = jnp.where(
            jnp.logical_or(j < NBLK - 1, col < MEM - (NBLK - 1) * CBLK),
            raw, -1e9)
    else:
        sims = raw

    @pl.when(jnp.logical_and(p == 0, j == 0))
    def _():
        pos_ref[...] = jnp.full((B, 1), -1e9, jnp.float32)
        neg_ref[...] = jnp.full((B, 1), -1e9, jnp.float32)
        cnt_ref[...] = jnp.zeros((B, 1), jnp.float32)

    @pl.when(p == 0)
    def _():
        memv = v_ref[0]
        match = y_ref[...] == memv
        posb = jnp.max(jnp.where(match, sims, -1e9), axis=1, keepdims=True)
        negb = jnp.max(jnp.where(match, -1e9, sims), axis=1, keepdims=True)
        pos_ref[...] = jnp.maximum(pos_ref[...], posb)
        neg_ref[...] = jnp.maximum(neg_ref[...], negb)

    @pl.when(p == 1)
    def _():
        gt = sims > pos_ref[...]
        cnt_ref[...] += jnp.sum(gt.astype(jnp.float32), axis=1, keepdims=True)

    @pl.when(jnp.logical_and(p == 1, j == NBLK - 1))
    def _():
        pos = pos_ref[...]
        neg = neg_ref[...]
        cnt = cnt_ref[...]
        loss_ref[...] = jnp.mean(
            jnp.maximum(neg - pos + ALPHA, 0.0)).reshape(1, 1)
        acc_ref[...] = jnp.mean((cnt < 1.0).astype(jnp.float32)).reshape(1, 1)
        ii = lax.broadcasted_iota(jnp.int32, (1, 3), 1).astype(jnp.float32)
        thr = (ii + 1.0) * (ii + 1.0) + 1.0
        oth_ref[...] = jnp.mean((cnt < thr).astype(jnp.float32), axis=0,
                                keepdims=True)


def _mem_stats(y2, q, kt, v3):
    grid = (2, NBLK)
    return pl.pallas_call(
        _mem_body,
        grid=grid,
        in_specs=[
            pl.BlockSpec((B, 1), lambda p, j: (0, 0)),
            pl.BlockSpec((B, KEY), lambda p, j: (0, 0)),
            pl.BlockSpec((CBLK, KEY), lambda p, j: (j, 0)),
            pl.BlockSpec((1, 1, CBLK), lambda p, j: (j, 0, 0)),
        ],
        out_specs=[
            pl.BlockSpec((1, 1), lambda p, j: (0, 0)),
            pl.BlockSpec((1, 1), lambda p, j: (0, 0)),
            pl.BlockSpec((1, 3), lambda p, j: (0, 0)),
        ],
        out_shape=[
            jax.ShapeDtypeStruct((1, 1), jnp.float32),
            jax.ShapeDtypeStruct((1, 1), jnp.float32),
            jax.ShapeDtypeStruct((1, 3), jnp.float32),
        ],
        scratch_shapes=[
            pltpu.VMEM((B, 1), jnp.float32),
            pltpu.VMEM((B, 1), jnp.float32),
            pltpu.VMEM((B, 1), jnp.float32),
        ],
        compiler_params=pltpu.CompilerParams(
            dimension_semantics=("arbitrary", "arbitrary")),
    )(y2, q, kt, v3)


def kernel(x, y, embed, conv_w3, conv_b3, conv_w4, conv_b4, conv_w5, conv_b5,
           fc_w, fc_b, mem_K, mem_V):
    xf = x.reshape(B * L).astype(jnp.int32)
    pad_ch = KNP - KN
    w3p = jnp.pad(conv_w3, ((0, 0), (0, 0), (0, pad_ch)))
    w4p = jnp.pad(conv_w4, ((0, 0), (0, 0), (0, pad_ch)))
    w5p = jnp.pad(conv_w5, ((0, 0), (0, 0), (0, pad_ch)))
    wcat = jnp.concatenate(
        [w.transpose(1, 0, 2).reshape(D, -1) for w in (w3p, w4p, w5p)],
        axis=1)
    b3p = jnp.pad(conv_b3, (0, pad_ch)).reshape(1, KNP)
    b4p = jnp.pad(conv_b4, (0, pad_ch)).reshape(1, KNP)
    b5p = jnp.pad(conv_b5, (0, pad_ch)).reshape(1, KNP)
    fcw = fc_w.reshape(3, KN, KEY)
    fcwp = jnp.pad(fcw, ((0, 0), (0, pad_ch), (0, 0))).reshape(3 * KNP, KEY)
    fcbp = fc_b.reshape(1, KEY)
    kt = jnp.pad(mem_K, ((0, MEMP - MEM), (0, 0)))
    v3 = jnp.pad(mem_V.astype(jnp.int32), (0, MEMP - MEM),
                 constant_values=-1).reshape(NBLK, 1, CBLK)
    y2 = y.astype(jnp.int32).reshape(B, 1)

    e = _sc_gather(embed, xf)
    e3 = e.reshape(B, L, D)

    q = _q_from_e(e3, wcat, b3p, b4p, b5p, fcwp, fcbp)

    loss2, acc2, oth2 = _mem_stats(y2, q, kt, v3)
    return (loss2.reshape(()), acc2.reshape(()), oth2.reshape(3))


# speedup vs baseline: 109.6029x; 1.0731x over previous
import functools

import jax
import jax.numpy as jnp
from jax import lax
from jax.experimental import pallas as pl
from jax.experimental.pallas import tpu as pltpu
from jax.experimental.pallas import tpu_sc as plsc

B = 1024
L = 50
D = 128
KEY = 128
KN = 100
KNP = 128
MEM = 100000
CBLK = 4000
NBLK = 25
BT = 64
ALPHA = 0.1

NW = 32
GCH = 320


def _sc_gather(embed, idx_flat):
    n = idx_flat.shape[0]
    bpw = n // NW
    mesh = plsc.VectorSubcoreMesh(core_axis_name="c", subcore_axis_name="s")

    @functools.partial(
        pl.kernel,
        mesh=mesh,
        out_type=jax.ShapeDtypeStruct((n, D), jnp.float32),
        scratch_types=[
            pltpu.VMEM((GCH,), jnp.int32),
            pltpu.VMEM((GCH, D), jnp.float32),
            pltpu.SemaphoreType.DMA,
        ],
    )
    def gather_kernel(table_hbm, idx_hbm, out_hbm, idx_v, rows_v, sem):
        wid = lax.axis_index("s") * 2 + lax.axis_index("c")
        base = wid * bpw

        @pl.loop(0, bpw, step=GCH)
        def _(off):
            pltpu.sync_copy(idx_hbm.at[pl.ds(base + off, GCH)], idx_v)
            pltpu.async_copy(table_hbm.at[idx_v], rows_v, sem).wait()
            pltpu.sync_copy(rows_v, out_hbm.at[pl.ds(base + off, GCH)])

    return gather_kernel(embed, idx_flat)


def _q_body(e_ref, wcat_ref, b3_ref, b4_ref, b5_ref, fcw_ref, fcb_ref,
            q_ref):
    e2 = e_ref[...].reshape(BT * L, D)
    g = jnp.dot(e2, wcat_ref[...],
                preferred_element_type=jnp.float32)
    g3 = g.reshape(BT, L, 12 * KNP)
    feats = []
    off = 0
    for b_ref, s in ((b3_ref, 3), (b4_ref, 4), (b5_ref, 5)):
        t = L - s + 1
        acc = None
        for dt in range(s):
            sl = g3[:, dt:dt + t, off:off + KNP]
            acc = sl if acc is None else acc + sl
            off += KNP
        c = jnp.maximum(acc + b_ref[...][None, :, :], 0.0)
        feats.append(jnp.max(c, axis=1))
    q = jnp.zeros((BT, KEY), jnp.float32) + fcb_ref[...]
    for i, f in enumerate(feats):
        q = q + jnp.dot(f, fcw_ref[pl.ds(i * KNP, KNP), :],
                        preferred_element_type=jnp.float32)
    norm = jnp.sqrt(jnp.sum(q * q, axis=1, keepdims=True))
    q_ref[...] = q / (norm + 1e-8)


def _q_from_e(e3, wcat, b3p, b4p, b5p, fcwp, fcbp):
    grid = (B // BT,)
    return pl.pallas_call(
        _q_body,
        grid=grid,
        in_specs=[
            pl.BlockSpec((BT, L, D), lambda i: (i, 0, 0)),
            pl.BlockSpec((D, 12 * KNP), lambda i: (0, 0)),
            pl.BlockSpec((1, KNP), lambda i: (0, 0)),
            pl.BlockSpec((1, KNP), lambda i: (0, 0)),
            pl.BlockSpec((1, KNP), lambda i: (0, 0)),
            pl.BlockSpec((3 * KNP, KEY), lambda i: (0, 0)),
            pl.BlockSpec((1, KEY), lambda i: (0, 0)),
        ],
        out_specs=pl.BlockSpec((BT, KEY), lambda i: (i, 0)),
        out_shape=jax.ShapeDtypeStruct((B, KEY), jnp.float32),
    )(e3, wcat, b3p, b4p, b5p, fcwp, fcbp)


def _mem_body(y_ref, q_ref, kt_ref, v_ref, loss_ref, acc_ref, oth_ref,
              pos_ref, neg_ref, cnt_ref):
    p = pl.program_id(0)
    j = pl.program_id(1)
    sims = lax.dot_general(q_ref[...], kt_ref[...],
                           (((1,), (1,)), ((), ())),
                           preferred_element_type=jnp.float32)

    @pl.when(jnp.logical_and(p == 0, j == 0))
    def _():
        pos_ref[...] = jnp.full((B, 1), -1e9, jnp.float32)
        neg_ref[...] = jnp.full((B, 1), -1e9, jnp.float32)
        cnt_ref[...] = jnp.zeros((B, 1), jnp.float32)

    @pl.when(p == 0)
    def _():
        memv = v_ref[0]
        match = y_ref[...] == memv
        posb = jnp.max(jnp.where(match, sims, -1e9), axis=1, keepdims=True)
        negb = jnp.max(jnp.where(match, -1e9, sims), axis=1, keepdims=True)
        pos_ref[...] = jnp.maximum(pos_ref[...], posb)
        neg_ref[...] = jnp.maximum(neg_ref[...], negb)

    @pl.when(p == 1)
    def _():
        gt = sims > pos_ref[...]
        cnt_ref[...] += jnp.sum(gt.astype(jnp.float32), axis=1, keepdims=True)

    @pl.when(jnp.logical_and(p == 1, j == NBLK - 1))
    def _():
        pos = pos_ref[...]
        neg = neg_ref[...]
        cnt = cnt_ref[...]
        loss_ref[...] = jnp.mean(
            jnp.maximum(neg - pos + ALPHA, 0.0)).reshape(1, 1)
        acc_ref[...] = jnp.mean((cnt < 1.0).astype(jnp.float32)).reshape(1, 1)
        ii = lax.broadcasted_iota(jnp.int32, (1, 3), 1).astype(jnp.float32)
        thr = (ii + 1.0) * (ii + 1.0) + 1.0
        oth_ref[...] = jnp.mean((cnt < thr).astype(jnp.float32), axis=0,
                                keepdims=True)


def _mem_stats(y2, q, kt, v3):
    grid = (2, NBLK)
    return pl.pallas_call(
        _mem_body,
        grid=grid,
        in_specs=[
            pl.BlockSpec((B, 1), lambda p, j: (0, 0)),
            pl.BlockSpec((B, KEY), lambda p, j: (0, 0)),
            pl.BlockSpec((CBLK, KEY), lambda p, j: (j, 0)),
            pl.BlockSpec((1, 1, CBLK), lambda p, j: (j, 0, 0)),
        ],
        out_specs=[
            pl.BlockSpec((1, 1), lambda p, j: (0, 0)),
            pl.BlockSpec((1, 1), lambda p, j: (0, 0)),
            pl.BlockSpec((1, 3), lambda p, j: (0, 0)),
        ],
        out_shape=[
            jax.ShapeDtypeStruct((1, 1), jnp.float32),
            jax.ShapeDtypeStruct((1, 1), jnp.float32),
            jax.ShapeDtypeStruct((1, 3), jnp.float32),
        ],
        scratch_shapes=[
            pltpu.VMEM((B, 1), jnp.float32),
            pltpu.VMEM((B, 1), jnp.float32),
            pltpu.VMEM((B, 1), jnp.float32),
        ],
        compiler_params=pltpu.CompilerParams(
            dimension_semantics=("arbitrary", "arbitrary")),
    )(y2, q, kt, v3)


def kernel(x, y, embed, conv_w3, conv_b3, conv_w4, conv_b4, conv_w5, conv_b5,
           fc_w, fc_b, mem_K, mem_V):
    xf = x.reshape(B * L).astype(jnp.int32)
    pad_ch = KNP - KN
    w3p = jnp.pad(conv_w3, ((0, 0), (0, 0), (0, pad_ch)))
    w4p = jnp.pad(conv_w4, ((0, 0), (0, 0), (0, pad_ch)))
    w5p = jnp.pad(conv_w5, ((0, 0), (0, 0), (0, pad_ch)))
    wcat = jnp.concatenate(
        [w.transpose(1, 0, 2).reshape(D, -1) for w in (w3p, w4p, w5p)],
        axis=1)
    b3p = jnp.pad(conv_b3, (0, pad_ch)).reshape(1, KNP)
    b4p = jnp.pad(conv_b4, (0, pad_ch)).reshape(1, KNP)
    b5p = jnp.pad(conv_b5, (0, pad_ch)).reshape(1, KNP)
    fcw = fc_w.reshape(3, KN, KEY)
    fcwp = jnp.pad(fcw, ((0, 0), (0, pad_ch), (0, 0))).reshape(3 * KNP, KEY)
    fcbp = fc_b.reshape(1, KEY)
    kt = mem_K
    v3 = mem_V.astype(jnp.int32).reshape(NBLK, 1, CBLK)
    y2 = y.astype(jnp.int32).reshape(B, 1)

    e = _sc_gather(embed, xf)
    e3 = e.reshape(B, L, D)

    q = _q_from_e(e3, wcat, b3p, b4p, b5p, fcwp, fcbp)

    loss2, acc2, oth2 = _mem_stats(y2, q, kt, v3)
    return (loss2.reshape(()), acc2.reshape(()), oth2.reshape(3))


# speedup vs baseline: 141.9897x; 1.2955x over previous
import functools

import jax
import jax.numpy as jnp
from jax import lax
from jax.experimental import pallas as pl
from jax.experimental.pallas import tpu as pltpu
from jax.experimental.pallas import tpu_sc as plsc

B = 1024
L = 50
D = 128
KEY = 128
KN = 100
KNP = 128
MEM = 100000
CBLK = 4000
NBLK = 25
BT = 64
ALPHA = 0.1

NW = 32
GCH = 320


def _sc_gather(embed, idx_flat):
    n = idx_flat.shape[0]
    bpw = n // NW
    mesh = plsc.VectorSubcoreMesh(core_axis_name="c", subcore_axis_name="s")

    @functools.partial(
        pl.kernel,
        mesh=mesh,
        out_type=jax.ShapeDtypeStruct((n, D), jnp.float32),
        scratch_types=[
            pltpu.VMEM((GCH,), jnp.int32),
            pltpu.VMEM((GCH, D), jnp.float32),
            pltpu.SemaphoreType.DMA,
        ],
    )
    def gather_kernel(table_hbm, idx_hbm, out_hbm, idx_v, rows_v, sem):
        wid = lax.axis_index("s") * 2 + lax.axis_index("c")
        base = wid * bpw

        @pl.loop(0, bpw, step=GCH)
        def _(off):
            pltpu.sync_copy(idx_hbm.at[pl.ds(base + off, GCH)], idx_v)
            pltpu.async_copy(table_hbm.at[idx_v], rows_v, sem).wait()
            pltpu.sync_copy(rows_v, out_hbm.at[pl.ds(base + off, GCH)])

    return gather_kernel(embed, idx_flat)


def _q_body(e_ref, wp_ref, ws_ref, b3_ref, b4_ref, b5_ref, fcw_ref, fcb_ref,
            q_ref):
    e2 = e_ref[...].reshape(BT * L, D)
    e2n = jnp.concatenate([e2[1:], jnp.zeros((1, D), jnp.float32)], axis=0)
    ee = jnp.concatenate([e2, e2n], axis=1)
    gp = jnp.dot(ee, wp_ref[...],
                 preferred_element_type=jnp.float32)
    gs = jnp.dot(e2, ws_ref[...],
                 preferred_element_type=jnp.float32)
    p3 = gp.reshape(BT, L, 5 * KNP)
    s3 = gs.reshape(BT, L, 2 * KNP)
    c3 = p3[:, 0:48, 0 * KNP:1 * KNP] + s3[:, 2:50, 0 * KNP:1 * KNP]
    c4 = p3[:, 0:47, 1 * KNP:2 * KNP] + p3[:, 2:49, 2 * KNP:3 * KNP]
    c5 = (p3[:, 0:46, 3 * KNP:4 * KNP] + p3[:, 2:48, 4 * KNP:5 * KNP]
          + s3[:, 4:50, 1 * KNP:2 * KNP])
    feats = []
    for c, b_ref in ((c3, b3_ref), (c4, b4_ref), (c5, b5_ref)):
        feats.append(jnp.maximum(jnp.max(c, axis=1) + b_ref[...], 0.0))
    q = jnp.zeros((BT, KEY), jnp.float32) + fcb_ref[...]
    for i, f in enumerate(feats):
        q = q + jnp.dot(f, fcw_ref[pl.ds(i * KNP, KNP), :],
                        preferred_element_type=jnp.float32)
    norm = jnp.sqrt(jnp.sum(q * q, axis=1, keepdims=True))
    q_ref[...] = q / (norm + 1e-8)


def _q_from_e(e3, wp, ws, b3p, b4p, b5p, fcwp, fcbp):
    grid = (B // BT,)
    return pl.pallas_call(
        _q_body,
        grid=grid,
        in_specs=[
            pl.BlockSpec((BT, L, D), lambda i: (i, 0, 0)),
            pl.BlockSpec((2 * D, 5 * KNP), lambda i: (0, 0)),
            pl.BlockSpec((D, 2 * KNP), lambda i: (0, 0)),
            pl.BlockSpec((1, KNP), lambda i: (0, 0)),
            pl.BlockSpec((1, KNP), lambda i: (0, 0)),
            pl.BlockSpec((1, KNP), lambda i: (0, 0)),
            pl.BlockSpec((3 * KNP, KEY), lambda i: (0, 0)),
            pl.BlockSpec((1, KEY), lambda i: (0, 0)),
        ],
        out_specs=pl.BlockSpec((BT, KEY), lambda i: (i, 0)),
        out_shape=jax.ShapeDtypeStruct((B, KEY), jnp.float32),
    )(e3, wp, ws, b3p, b4p, b5p, fcwp, fcbp)


def _mem_body(y_ref, q_ref, kt_ref, v_ref, loss_ref, acc_ref, oth_ref,
              pos_ref, neg_ref, cnt_ref):
    p = pl.program_id(0)
    j = pl.program_id(1)
    sims = lax.dot_general(q_ref[...], kt_ref[...],
                           (((1,), (1,)), ((), ())),
                           preferred_element_type=jnp.float32)

    @pl.when(jnp.logical_and(p == 0, j == 0))
    def _():
        pos_ref[...] = jnp.full((B, 1), -1e9, jnp.float32)
        neg_ref[...] = jnp.full((B, 1), -1e9, jnp.float32)
        cnt_ref[...] = jnp.zeros((B, 1), jnp.float32)

    @pl.when(p == 0)
    def _():
        memv = v_ref[0]
        match = y_ref[...] == memv
        posb = jnp.max(jnp.where(match, sims, -1e9), axis=1, keepdims=True)
        negb = jnp.max(jnp.where(match, -1e9, sims), axis=1, keepdims=True)
        pos_ref[...] = jnp.maximum(pos_ref[...], posb)
        neg_ref[...] = jnp.maximum(neg_ref[...], negb)

    @pl.when(p == 1)
    def _():
        gt = sims > pos_ref[...]
        cnt_ref[...] += jnp.sum(gt.astype(jnp.float32), axis=1, keepdims=True)

    @pl.when(jnp.logical_and(p == 1, j == NBLK - 1))
    def _():
        pos = pos_ref[...]
        neg = neg_ref[...]
        cnt = cnt_ref[...]
        loss_ref[...] = jnp.mean(
            jnp.maximum(neg - pos + ALPHA, 0.0)).reshape(1, 1)
        acc_ref[...] = jnp.mean((cnt < 1.0).astype(jnp.float32)).reshape(1, 1)
        ii = lax.broadcasted_iota(jnp.int32, (1, 3), 1).astype(jnp.float32)
        thr = (ii + 1.0) * (ii + 1.0) + 1.0
        oth_ref[...] = jnp.mean((cnt < thr).astype(jnp.float32), axis=0,
                                keepdims=True)


def _mem_stats(y2, q, kt, v3):
    grid = (2, NBLK)
    return pl.pallas_call(
        _mem_body,
        grid=grid,
        in_specs=[
            pl.BlockSpec((B, 1), lambda p, j: (0, 0)),
            pl.BlockSpec((B, KEY), lambda p, j: (0, 0)),
            pl.BlockSpec((CBLK, KEY), lambda p, j: (j, 0)),
            pl.BlockSpec((1, 1, CBLK), lambda p, j: (j, 0, 0)),
        ],
        out_specs=[
            pl.BlockSpec((1, 1), lambda p, j: (0, 0)),
            pl.BlockSpec((1, 1), lambda p, j: (0, 0)),
            pl.BlockSpec((1, 3), lambda p, j: (0, 0)),
        ],
        out_shape=[
            jax.ShapeDtypeStruct((1, 1), jnp.float32),
            jax.ShapeDtypeStruct((1, 1), jnp.float32),
            jax.ShapeDtypeStruct((1, 3), jnp.float32),
        ],
        scratch_shapes=[
            pltpu.VMEM((B, 1), jnp.float32),
            pltpu.VMEM((B, 1), jnp.float32),
            pltpu.VMEM((B, 1), jnp.float32),
        ],
        compiler_params=pltpu.CompilerParams(
            dimension_semantics=("arbitrary", "arbitrary")),
    )(y2, q, kt, v3)


def kernel(x, y, embed, conv_w3, conv_b3, conv_w4, conv_b4, conv_w5, conv_b5,
           fc_w, fc_b, mem_K, mem_V):
    xf = x.reshape(B * L).astype(jnp.int32)
    pad_ch = KNP - KN
    w3p = jnp.pad(conv_w3, ((0, 0), (0, 0), (0, pad_ch)))
    w4p = jnp.pad(conv_w4, ((0, 0), (0, 0), (0, pad_ch)))
    w5p = jnp.pad(conv_w5, ((0, 0), (0, 0), (0, pad_ch)))
    def _pair(w, a):
        return jnp.concatenate([w[a], w[a + 1]], axis=0)
    wp = jnp.concatenate(
        [_pair(w3p, 0), _pair(w4p, 0), _pair(w4p, 2),
         _pair(w5p, 0), _pair(w5p, 2)], axis=1)
    ws = jnp.concatenate([w3p[2], w5p[4]], axis=1)
    b3p = jnp.pad(conv_b3, (0, pad_ch)).reshape(1, KNP)
    b4p = jnp.pad(conv_b4, (0, pad_ch)).reshape(1, KNP)
    b5p = jnp.pad(conv_b5, (0, pad_ch)).reshape(1, KNP)
    fcw = fc_w.reshape(3, KN, KEY)
    fcwp = jnp.pad(fcw, ((0, 0), (0, pad_ch), (0, 0))).reshape(3 * KNP, KEY)
    fcbp = fc_b.reshape(1, KEY)
    kt = mem_K
    v3 = mem_V.astype(jnp.int32).reshape(NBLK, 1, CBLK)
    y2 = y.astype(jnp.int32).reshape(B, 1)

    e = _sc_gather(embed, xf)
    e3 = e.reshape(B, L, D)

    q = _q_from_e(e3, wp, ws, b3p, b4p, b5p, fcwp, fcbp)

    loss2, acc2, oth2 = _mem_stats(y2, q, kt, v3)
    return (loss2.reshape(()), acc2.reshape(()), oth2.reshape(3))


# speedup vs baseline: 142.8929x; 1.0064x over previous
import functools

import jax
import jax.numpy as jnp
from jax import lax
from jax.experimental import pallas as pl
from jax.experimental.pallas import tpu as pltpu
from jax.experimental.pallas import tpu_sc as plsc

B = 1024
L = 50
D = 128
KEY = 128
KN = 100
KNP = 128
MEM = 100000
CBLK = 4000
NBLK = 25
BT = 128
ALPHA = 0.1

NW = 32
GCH = 400


def _sc_gather(embed, idx_flat):
    n = idx_flat.shape[0]
    bpw = n // NW
    mesh = plsc.VectorSubcoreMesh(core_axis_name="c", subcore_axis_name="s")

    @functools.partial(
        pl.kernel,
        mesh=mesh,
        out_type=jax.ShapeDtypeStruct((n, D), jnp.float32),
        scratch_types=[
            pltpu.VMEM((GCH,), jnp.int32),
            pltpu.VMEM((GCH, D), jnp.float32),
            pltpu.SemaphoreType.DMA,
        ],
    )
    def gather_kernel(table_hbm, idx_hbm, out_hbm, idx_v, rows_v, sem):
        wid = lax.axis_index("s") * 2 + lax.axis_index("c")
        base = wid * bpw

        @pl.loop(0, bpw, step=GCH)
        def _(off):
            pltpu.sync_copy(idx_hbm.at[pl.ds(base + off, GCH)], idx_v)
            pltpu.async_copy(table_hbm.at[idx_v], rows_v, sem).wait()
            pltpu.sync_copy(rows_v, out_hbm.at[pl.ds(base + off, GCH)])

    return gather_kernel(embed, idx_flat)


def _q_body(e_ref, wp_ref, ws_ref, b3_ref, b4_ref, b5_ref, fcw_ref, fcb_ref,
            q_ref):
    e2 = e_ref[...].reshape(BT * L, D)
    e2n = jnp.concatenate([e2[1:], jnp.zeros((1, D), jnp.float32)], axis=0)
    ee = jnp.concatenate([e2, e2n], axis=1)
    gp = jnp.dot(ee, wp_ref[...],
                 preferred_element_type=jnp.float32)
    gs = jnp.dot(e2, ws_ref[...],
                 preferred_element_type=jnp.float32)
    p3 = gp.reshape(BT, L, 5 * KNP)
    s3 = gs.reshape(BT, L, 2 * KNP)
    c3 = p3[:, 0:48, 0 * KNP:1 * KNP] + s3[:, 2:50, 0 * KNP:1 * KNP]
    c4 = p3[:, 0:47, 1 * KNP:2 * KNP] + p3[:, 2:49, 2 * KNP:3 * KNP]
    c5 = (p3[:, 0:46, 3 * KNP:4 * KNP] + p3[:, 2:48, 4 * KNP:5 * KNP]
          + s3[:, 4:50, 1 * KNP:2 * KNP])
    feats = []
    for c, b_ref in ((c3, b3_ref), (c4, b4_ref), (c5, b5_ref)):
        feats.append(jnp.maximum(jnp.max(c, axis=1) + b_ref[...], 0.0))
    q = jnp.zeros((BT, KEY), jnp.float32) + fcb_ref[...]
    for i, f in enumerate(feats):
        q = q + jnp.dot(f, fcw_ref[pl.ds(i * KNP, KNP), :],
                        preferred_element_type=jnp.float32)
    norm = jnp.sqrt(jnp.sum(q * q, axis=1, keepdims=True))
    q_ref[...] = q / (norm + 1e-8)


def _q_from_e(e3, wp, ws, b3p, b4p, b5p, fcwp, fcbp):
    nb = e3.shape[0]
    grid = (nb // BT,)
    return pl.pallas_call(
        _q_body,
        grid=grid,
        in_specs=[
            pl.BlockSpec((BT, L, D), lambda i: (i, 0, 0)),
            pl.BlockSpec((2 * D, 5 * KNP), lambda i: (0, 0)),
            pl.BlockSpec((D, 2 * KNP), lambda i: (0, 0)),
            pl.BlockSpec((1, KNP), lambda i: (0, 0)),
            pl.BlockSpec((1, KNP), lambda i: (0, 0)),
            pl.BlockSpec((1, KNP), lambda i: (0, 0)),
            pl.BlockSpec((3 * KNP, KEY), lambda i: (0, 0)),
            pl.BlockSpec((1, KEY), lambda i: (0, 0)),
        ],
        out_specs=pl.BlockSpec((BT, KEY), lambda i: (i, 0)),
        out_shape=jax.ShapeDtypeStruct((nb, KEY), jnp.float32),
    )(e3, wp, ws, b3p, b4p, b5p, fcwp, fcbp)


def _mem_body(y_ref, q_ref, kt_ref, v_ref, loss_ref, acc_ref, oth_ref,
              pos_ref, neg_ref, cnt_ref):
    p = pl.program_id(0)
    j = pl.program_id(1)
    sims = lax.dot_general(q_ref[...], kt_ref[...],
                           (((1,), (1,)), ((), ())),
                           preferred_element_type=jnp.float32)

    @pl.when(jnp.logical_and(p == 0, j == 0))
    def _():
        pos_ref[...] = jnp.full((B, 1), -1e9, jnp.float32)
        neg_ref[...] = jnp.full((B, 1), -1e9, jnp.float32)
        cnt_ref[...] = jnp.zeros((B, 1), jnp.float32)

    @pl.when(p == 0)
    def _():
        memv = v_ref[0]
        match = y_ref[...] == memv
        posb = jnp.max(jnp.where(match, sims, -1e9), axis=1, keepdims=True)
        negb = jnp.max(jnp.where(match, -1e9, sims), axis=1, keepdims=True)
        pos_ref[...] = jnp.maximum(pos_ref[...], posb)
        neg_ref[...] = jnp.maximum(neg_ref[...], negb)

    @pl.when(p == 1)
    def _():
        gt = sims > pos_ref[...]
        cnt_ref[...] += jnp.sum(gt.astype(jnp.float32), axis=1, keepdims=True)

    @pl.when(jnp.logical_and(p == 1, j == NBLK - 1))
    def _():
        pos = pos_ref[...]
        neg = neg_ref[...]
        cnt = cnt_ref[...]
        loss_ref[...] = jnp.mean(
            jnp.maximum(neg - pos + ALPHA, 0.0)).reshape(1, 1)
        acc_ref[...] = jnp.mean((cnt < 1.0).astype(jnp.float32)).reshape(1, 1)
        ii = lax.broadcasted_iota(jnp.int32, (1, 3), 1).astype(jnp.float32)
        thr = (ii + 1.0) * (ii + 1.0) + 1.0
        oth_ref[...] = jnp.mean((cnt < thr).astype(jnp.float32), axis=0,
                                keepdims=True)


def _mem_stats(y2, q, kt, v3):
    grid = (2, NBLK)
    return pl.pallas_call(
        _mem_body,
        grid=grid,
        in_specs=[
            pl.BlockSpec((B, 1), lambda p, j: (0, 0)),
            pl.BlockSpec((B, KEY), lambda p, j: (0, 0)),
            pl.BlockSpec((CBLK, KEY), lambda p, j: (j, 0)),
            pl.BlockSpec((1, 1, CBLK), lambda p, j: (j, 0, 0)),
        ],
        out_specs=[
            pl.BlockSpec((1, 1), lambda p, j: (0, 0)),
            pl.BlockSpec((1, 1), lambda p, j: (0, 0)),
            pl.BlockSpec((1, 3), lambda p, j: (0, 0)),
        ],
        out_shape=[
            jax.ShapeDtypeStruct((1, 1), jnp.float32),
            jax.ShapeDtypeStruct((1, 1), jnp.float32),
            jax.ShapeDtypeStruct((1, 3), jnp.float32),
        ],
        scratch_shapes=[
            pltpu.VMEM((B, 1), jnp.float32),
            pltpu.VMEM((B, 1), jnp.float32),
            pltpu.VMEM((B, 1), jnp.float32),
        ],
        compiler_params=pltpu.CompilerParams(
            dimension_semantics=("arbitrary", "arbitrary")),
    )(y2, q, kt, v3)


def kernel(x, y, embed, conv_w3, conv_b3, conv_w4, conv_b4, conv_w5, conv_b5,
           fc_w, fc_b, mem_K, mem_V):
    xf = x.reshape(B * L).astype(jnp.int32)
    pad_ch = KNP - KN
    w3p = jnp.pad(conv_w3, ((0, 0), (0, 0), (0, pad_ch)))
    w4p = jnp.pad(conv_w4, ((0, 0), (0, 0), (0, pad_ch)))
    w5p = jnp.pad(conv_w5, ((0, 0), (0, 0), (0, pad_ch)))
    def _pair(w, a):
        return jnp.concatenate([w[a], w[a + 1]], axis=0)
    wp = jnp.concatenate(
        [_pair(w3p, 0), _pair(w4p, 0), _pair(w4p, 2),
         _pair(w5p, 0), _pair(w5p, 2)], axis=1)
    ws = jnp.concatenate([w3p[2], w5p[4]], axis=1)
    b3p = jnp.pad(conv_b3, (0, pad_ch)).reshape(1, KNP)
    b4p = jnp.pad(conv_b4, (0, pad_ch)).reshape(1, KNP)
    b5p = jnp.pad(conv_b5, (0, pad_ch)).reshape(1, KNP)
    fcw = fc_w.reshape(3, KN, KEY)
    fcwp = jnp.pad(fcw, ((0, 0), (0, pad_ch), (0, 0))).reshape(3 * KNP, KEY)
    fcbp = fc_b.reshape(1, KEY)
    kt = mem_K
    v3 = mem_V.astype(jnp.int32).reshape(NBLK, 1, CBLK)
    y2 = y.astype(jnp.int32).reshape(B, 1)

    half = (B // 2) * L
    e1 = _sc_gather(embed, xf[:half])
    e2h = _sc_gather(embed, xf[half:])
    q1 = _q_from_e(e1.reshape(B // 2, L, D), wp, ws, b3p, b4p, b5p,
                   fcwp, fcbp)
    q2 = _q_from_e(e2h.reshape(B // 2, L, D), wp, ws, b3p, b4p, b5p,
                   fcwp, fcbp)
    q = jnp.concatenate([q1, q2], axis=0)

    loss2, acc2, oth2 = _mem_stats(y2, q, kt, v3)
    return (loss2.reshape(()), acc2.reshape(()), oth2.reshape(3))


# speedup vs baseline: 144.4475x; 1.0109x over previous
import functools

import jax
import jax.numpy as jnp
from jax import lax
from jax.experimental import pallas as pl
from jax.experimental.pallas import tpu as pltpu
from jax.experimental.pallas import tpu_sc as plsc

B = 1024
L = 50
D = 128
KEY = 128
KN = 100
KNP = 128
MEM = 100000
CBLK = 4096
NBLK = 25
BT = 128
ALPHA = 0.1

NW = 32
GCH = 400


def _sc_gather(embed, idx_flat):
    n = idx_flat.shape[0]
    bpw = n // NW
    mesh = plsc.VectorSubcoreMesh(core_axis_name="c", subcore_axis_name="s")

    @functools.partial(
        pl.kernel,
        mesh=mesh,
        out_type=jax.ShapeDtypeStruct((n, D), jnp.float32),
        scratch_types=[
            pltpu.VMEM((GCH,), jnp.int32),
            pltpu.VMEM((GCH, D), jnp.float32),
            pltpu.SemaphoreType.DMA,
        ],
    )
    def gather_kernel(table_hbm, idx_hbm, out_hbm, idx_v, rows_v, sem):
        wid = lax.axis_index("s") * 2 + lax.axis_index("c")
        base = wid * bpw

        @pl.loop(0, bpw, step=GCH)
        def _(off):
            pltpu.sync_copy(idx_hbm.at[pl.ds(base + off, GCH)], idx_v)
            pltpu.async_copy(table_hbm.at[idx_v], rows_v, sem).wait()
            pltpu.sync_copy(rows_v, out_hbm.at[pl.ds(base + off, GCH)])

    return gather_kernel(embed, idx_flat)


def _q_body(e_ref, wp_ref, ws_ref, b3_ref, b4_ref, b5_ref, fcw_ref, fcb_ref,
            q_ref):
    e2 = e_ref[...].reshape(BT * L, D)
    e2n = jnp.concatenate([e2[1:], jnp.zeros((1, D), jnp.float32)], axis=0)
    ee = jnp.concatenate([e2, e2n], axis=1)
    gp = jnp.dot(ee, wp_ref[...],
                 preferred_element_type=jnp.float32)
    gs = jnp.dot(e2, ws_ref[...],
                 preferred_element_type=jnp.float32)
    p3 = gp.reshape(BT, L, 5 * KNP)
    s3 = gs.reshape(BT, L, 2 * KNP)
    c3 = p3[:, 0:48, 0 * KNP:1 * KNP] + s3[:, 2:50, 0 * KNP:1 * KNP]
    c4 = p3[:, 0:47, 1 * KNP:2 * KNP] + p3[:, 2:49, 2 * KNP:3 * KNP]
    c5 = (p3[:, 0:46, 3 * KNP:4 * KNP] + p3[:, 2:48, 4 * KNP:5 * KNP]
          + s3[:, 4:50, 1 * KNP:2 * KNP])
    feats = []
    for c, b_ref in ((c3, b3_ref), (c4, b4_ref), (c5, b5_ref)):
        feats.append(jnp.maximum(jnp.max(c, axis=1) + b_ref[...], 0.0))
    q = jnp.zeros((BT, KEY), jnp.float32) + fcb_ref[...]
    for i, f in enumerate(feats):
        q = q + jnp.dot(f, fcw_ref[pl.ds(i * KNP, KNP), :],
                        preferred_element_type=jnp.float32)
    norm = jnp.sqrt(jnp.sum(q * q, axis=1, keepdims=True))
    q_ref[...] = q / (norm + 1e-8)


def _q_from_e(e3, wp, ws, b3p, b4p, b5p, fcwp, fcbp):
    nb = e3.shape[0]
    grid = (nb // BT,)
    return pl.pallas_call(
        _q_body,
        grid=grid,
        in_specs=[
            pl.BlockSpec((BT, L, D), lambda i: (i, 0, 0)),
            pl.BlockSpec((2 * D, 5 * KNP), lambda i: (0, 0)),
            pl.BlockSpec((D, 2 * KNP), lambda i: (0, 0)),
            pl.BlockSpec((1, KNP), lambda i: (0, 0)),
            pl.BlockSpec((1, KNP), lambda i: (0, 0)),
            pl.BlockSpec((1, KNP), lambda i: (0, 0)),
            pl.BlockSpec((3 * KNP, KEY), lambda i: (0, 0)),
            pl.BlockSpec((1, KEY), lambda i: (0, 0)),
        ],
        out_specs=pl.BlockSpec((BT, KEY), lambda i: (i, 0)),
        out_shape=jax.ShapeDtypeStruct((nb, KEY), jnp.float32),
    )(e3, wp, ws, b3p, b4p, b5p, fcwp, fcbp)


def _mem_body(y_ref, q_ref, kt_ref, v_ref, loss_ref, acc_ref, oth_ref,
              pos_ref, neg_ref, cnt_ref):
    p = pl.program_id(0)
    j = pl.program_id(1)
    raw = lax.dot_general(q_ref[...], kt_ref[...],
                          (((1,), (1,)), ((), ())),
                          preferred_element_type=jnp.float32)
    col = j * CBLK + lax.broadcasted_iota(jnp.int32, (1, CBLK), 1)
    sims = jnp.where(col < MEM, raw, -1e9)

    @pl.when(jnp.logical_and(p == 0, j == 0))
    def _():
        pos_ref[...] = jnp.full((B, 1), -1e9, jnp.float32)
        neg_ref[...] = jnp.full((B, 1), -1e9, jnp.float32)
        cnt_ref[...] = jnp.zeros((B, 1), jnp.float32)

    @pl.when(p == 0)
    def _():
        memv = v_ref[0]
        match = y_ref[...] == memv
        posb = jnp.max(jnp.where(match, sims, -1e9), axis=1, keepdims=True)
        negb = jnp.max(jnp.where(match, -1e9, sims), axis=1, keepdims=True)
        pos_ref[...] = jnp.maximum(pos_ref[...], posb)
        neg_ref[...] = jnp.maximum(neg_ref[...], negb)

    @pl.when(p == 1)
    def _():
        gt = sims > pos_ref[...]
        cnt_ref[...] += jnp.sum(gt.astype(jnp.float32), axis=1, keepdims=True)

    @pl.when(jnp.logical_and(p == 1, j == NBLK - 1))
    def _():
        pos = pos_ref[...]
        neg = neg_ref[...]
        cnt = cnt_ref[...]
        loss_ref[...] = jnp.mean(
            jnp.maximum(neg - pos + ALPHA, 0.0)).reshape(1, 1)
        acc_ref[...] = jnp.mean((cnt < 1.0).astype(jnp.float32)).reshape(1, 1)
        ii = lax.broadcasted_iota(jnp.int32, (1, 3), 1).astype(jnp.float32)
        thr = (ii + 1.0) * (ii + 1.0) + 1.0
        oth_ref[...] = jnp.mean((cnt < thr).astype(jnp.float32), axis=0,
                                keepdims=True)


def _mem_stats(y2, q, kt, v3):
    grid = (2, NBLK)
    return pl.pallas_call(
        _mem_body,
        grid=grid,
        in_specs=[
            pl.BlockSpec((B, 1), lambda p, j: (0, 0)),
            pl.BlockSpec((B, KEY), lambda p, j: (0, 0)),
            pl.BlockSpec((CBLK, KEY), lambda p, j: (j, 0)),
            pl.BlockSpec((1, 1, CBLK), lambda p, j: (j, 0, 0)),
        ],
        out_specs=[
            pl.BlockSpec((1, 1), lambda p, j: (0, 0)),
            pl.BlockSpec((1, 1), lambda p, j: (0, 0)),
            pl.BlockSpec((1, 3), lambda p, j: (0, 0)),
        ],
        out_shape=[
            jax.ShapeDtypeStruct((1, 1), jnp.float32),
            jax.ShapeDtypeStruct((1, 1), jnp.float32),
            jax.ShapeDtypeStruct((1, 3), jnp.float32),
        ],
        scratch_shapes=[
            pltpu.VMEM((B, 1), jnp.float32),
            pltpu.VMEM((B, 1), jnp.float32),
            pltpu.VMEM((B, 1), jnp.float32),
        ],
        compiler_params=pltpu.CompilerParams(
            dimension_semantics=("arbitrary", "arbitrary")),
    )(y2, q, kt, v3)


def kernel(x, y, embed, conv_w3, conv_b3, conv_w4, conv_b4, conv_w5, conv_b5,
           fc_w, fc_b, mem_K, mem_V):
    xf = x.reshape(B * L).astype(jnp.int32)
    pad_ch = KNP - KN
    w3p = jnp.pad(conv_w3, ((0, 0), (0, 0), (0, pad_ch)))
    w4p = jnp.pad(conv_w4, ((0, 0), (0, 0), (0, pad_ch)))
    w5p = jnp.pad(conv_w5, ((0, 0), (0, 0), (0, pad_ch)))
    def _pair(w, a):
        return jnp.concatenate([w[a], w[a + 1]], axis=0)
    wp = jnp.concatenate(
        [_pair(w3p, 0), _pair(w4p, 0), _pair(w4p, 2),
         _pair(w5p, 0), _pair(w5p, 2)], axis=1)
    ws = jnp.concatenate([w3p[2], w5p[4]], axis=1)
    b3p = jnp.pad(conv_b3, (0, pad_ch)).reshape(1, KNP)
    b4p = jnp.pad(conv_b4, (0, pad_ch)).reshape(1, KNP)
    b5p = jnp.pad(conv_b5, (0, pad_ch)).reshape(1, KNP)
    fcw = fc_w.reshape(3, KN, KEY)
    fcwp = jnp.pad(fcw, ((0, 0), (0, pad_ch), (0, 0))).reshape(3 * KNP, KEY)
    fcbp = fc_b.reshape(1, KEY)
    kt = mem_K
    v3 = jnp.pad(mem_V.astype(jnp.int32), (0, NBLK * CBLK - MEM),
                 constant_values=-1).reshape(NBLK, 1, CBLK)
    y2 = y.astype(jnp.int32).reshape(B, 1)

    half = (B // 2) * L
    e1 = _sc_gather(embed, xf[:half])
    e2h = _sc_gather(embed, xf[half:])
    q1 = _q_from_e(e1.reshape(B // 2, L, D), wp, ws, b3p, b4p, b5p,
                   fcwp, fcbp)
    q2 = _q_from_e(e2h.reshape(B // 2, L, D), wp, ws, b3p, b4p, b5p,
                   fcwp, fcbp)
    q = jnp.concatenate([q1, q2], axis=0)

    loss2, acc2, oth2 = _mem_stats(y2, q, kt, v3)
    return (loss2.reshape(()), acc2.reshape(()), oth2.reshape(3))
